# baseline (speedup 1.0000x reference)
import functools

import jax
import jax.numpy as jnp
from jax import lax
from jax.experimental import pallas as pl

HIDDEN_DIM = 2048
NUM_EXPERTS = 16
TOP_K = 2
N_TOKENS = 16384

BLOCK_T = 1024


def _router_block(x_ref, w_ref, b_ref, logits_ref, weights_ref, idx_ref, mask_ref):
    x = x_ref[...]
    w = w_ref[...]
    b = b_ref[...]

    logits = lax.dot_general(
        x, w, dimension_numbers=(((1,), (1,)), ((), ())),
        preferred_element_type=jnp.float32,
    ) + b
    logits_ref[...] = logits

    t_sz = logits.shape[0]
    iota_e = lax.broadcasted_iota(jnp.int32, (t_sz, NUM_EXPERTS), 1)

    m1 = jnp.max(logits, axis=1, keepdims=True)
    i1 = jnp.min(jnp.where(logits == m1, iota_e, NUM_EXPERTS),
                 axis=1, keepdims=True)
    masked = jnp.where(iota_e == i1, -jnp.inf, logits)
    m2 = jnp.max(masked, axis=1, keepdims=True)
    i2 = jnp.min(jnp.where(masked == m2, iota_e, NUM_EXPERTS),
                 axis=1, keepdims=True)

    t = jnp.exp(m2 - m1)
    w1 = 1.0 / (1.0 + t)
    w2 = t * w1
    weights_ref[...] = jnp.concatenate([w1, w2], axis=1)
    idx_ref[...] = jnp.concatenate([i1, i2], axis=1)

    i1_t = jnp.transpose(i1)
    i2_t = jnp.transpose(i2)
    eids = lax.broadcasted_iota(jnp.int32, (NUM_EXPERTS, t_sz), 0)
    mask_ref[:, 0, :] = (eids == i1_t).astype(jnp.int32)
    mask_ref[:, 1, :] = (eids == i2_t).astype(jnp.int32)


@functools.partial(jax.jit, static_argnames=("interpret",))
def kernel(X, W, b, interpret=False):
    n_tokens = X.shape[0]
    grid = (n_tokens // BLOCK_T,)
    b2 = b.reshape(1, NUM_EXPERTS)

    out_shapes = (
        jax.ShapeDtypeStruct((n_tokens, NUM_EXPERTS), jnp.float32),
        jax.ShapeDtypeStruct((n_tokens, TOP_K), jnp.float32),
        jax.ShapeDtypeStruct((n_tokens, TOP_K), jnp.int32),
        jax.ShapeDtypeStruct((NUM_EXPERTS, TOP_K, n_tokens), jnp.int32),
    )
    in_specs = [
        pl.BlockSpec((BLOCK_T, HIDDEN_DIM), lambda i: (i, 0)),
        pl.BlockSpec((NUM_EXPERTS, HIDDEN_DIM), lambda i: (0, 0)),
        pl.BlockSpec((1, NUM_EXPERTS), lambda i: (0, 0)),
    ]
    out_specs = (
        pl.BlockSpec((BLOCK_T, NUM_EXPERTS), lambda i: (i, 0)),
        pl.BlockSpec((BLOCK_T, TOP_K), lambda i: (i, 0)),
        pl.BlockSpec((BLOCK_T, TOP_K), lambda i: (i, 0)),
        pl.BlockSpec((NUM_EXPERTS, TOP_K, BLOCK_T), lambda i: (0, 0, i)),
    )
    logits, weights, idx, mask = pl.pallas_call(
        _router_block,
        grid=grid,
        in_specs=in_specs,
        out_specs=out_specs,
        out_shape=out_shapes,
        interpret=interpret,
    )(X, W, b2)
    return (logits, weights, idx, mask)


# speedup vs baseline: 1.0797x; 1.0797x over previous
import functools

import jax
import jax.numpy as jnp
from jax import lax
from jax.experimental import pallas as pl

HIDDEN_DIM = 2048
NUM_EXPERTS = 16
TOP_K = 2
N_TOKENS = 16384

BLOCK_T = 1024


def _router_block(x_ref, w_ref, b_ref, logits_ref, weights_ref, idx_ref, mask_ref):
    x = x_ref[...]
    w = w_ref[...]
    b = b_ref[...]

    logits_t = lax.dot_general(
        w, x, dimension_numbers=(((1,), (1,)), ((), ())),
        preferred_element_type=jnp.float32,
    ) + b
    logits_ref[...] = jnp.transpose(logits_t)

    t_sz = logits_t.shape[1]
    iota_e = lax.broadcasted_iota(jnp.int32, (NUM_EXPERTS, t_sz), 0)

    m1 = jnp.max(logits_t, axis=0, keepdims=True)
    i1 = jnp.min(jnp.where(logits_t == m1, iota_e, NUM_EXPERTS),
                 axis=0, keepdims=True)
    masked = jnp.where(iota_e == i1, -jnp.inf, logits_t)
    m2 = jnp.max(masked, axis=0, keepdims=True)
    i2 = jnp.min(jnp.where(masked == m2, iota_e, NUM_EXPERTS),
                 axis=0, keepdims=True)

    t = jnp.exp(m2 - m1)
    w1 = 1.0 / (1.0 + t)
    w2 = t * w1
    weights_ref[...] = jnp.transpose(jnp.concatenate([w1, w2], axis=0))
    idx_ref[...] = jnp.transpose(jnp.concatenate([i1, i2], axis=0))

    mask_ref[:, 0, :] = (iota_e == i1).astype(jnp.int32)
    mask_ref[:, 1, :] = (iota_e == i2).astype(jnp.int32)


@functools.partial(jax.jit, static_argnames=("interpret",))
def kernel(X, W, b, interpret=False):
    n_tokens = X.shape[0]
    grid = (n_tokens // BLOCK_T,)
    b2 = b.reshape(NUM_EXPERTS, 1)

    out_shapes = (
        jax.ShapeDtypeStruct((n_tokens, NUM_EXPERTS), jnp.float32),
        jax.ShapeDtypeStruct((n_tokens, TOP_K), jnp.float32),
        jax.ShapeDtypeStruct((n_tokens, TOP_K), jnp.int32),
        jax.ShapeDtypeStruct((NUM_EXPERTS, TOP_K, n_tokens), jnp.int32),
    )
    in_specs = [
        pl.BlockSpec((BLOCK_T, HIDDEN_DIM), lambda i: (i, 0)),
        pl.BlockSpec((NUM_EXPERTS, HIDDEN_DIM), lambda i: (0, 0)),
        pl.BlockSpec((NUM_EXPERTS, 1), lambda i: (0, 0)),
    ]
    out_specs = (
        pl.BlockSpec((BLOCK_T, NUM_EXPERTS), lambda i: (i, 0)),
        pl.BlockSpec((BLOCK_T, TOP_K), lambda i: (i, 0)),
        pl.BlockSpec((BLOCK_T, TOP_K), lambda i: (i, 0)),
        pl.BlockSpec((NUM_EXPERTS, TOP_K, BLOCK_T), lambda i: (0, 0, i)),
    )
    logits, weights, idx, mask = pl.pallas_call(
        _router_block,
        grid=grid,
        in_specs=in_specs,
        out_specs=out_specs,
        out_shape=out_shapes,
        interpret=interpret,
    )(X, W, b2)
    return (logits, weights, idx, mask)
